# baseline (speedup 1.0000x reference)
import jax
import jax.numpy as jnp
from jax.experimental import pallas as pl
from jax.experimental.pallas import tpu as pltpu

T, H, F, E = 2048, 1024, 2048, 8
BT = 256
BF = 1024


def _moe_dense_kernel(logits_ref, x_ref, gate_ref, up_ref, down_ref, out_ref):
    e = pl.program_id(0)
    f = pl.program_id(1)
    t = pl.program_id(2)

    logits = logits_ref[...]
    probs = jax.nn.softmax(logits, axis=-1)
    m1 = jnp.max(probs, axis=-1, keepdims=True)
    cols = jax.lax.broadcasted_iota(jnp.int32, probs.shape, 1)
    i1 = jnp.argmax(probs, axis=-1)[:, None]
    masked = jnp.where(cols == i1, -jnp.inf, probs)
    m2 = jnp.max(masked, axis=-1, keepdims=True)
    i2 = jnp.argmax(masked, axis=-1)[:, None]
    denom = m1 + m2
    w_e = jnp.where(i1 == e, m1 / denom, 0.0) + jnp.where(i2 == e, m2 / denom, 0.0)

    x = x_ref[...]
    gate_w = gate_ref[0]
    up_w = up_ref[0]
    down_w = down_ref[0]
    dn = (((1,), (1,)), ((), ()))
    g = jax.lax.dot_general(x, gate_w, dn, preferred_element_type=jnp.float32)
    u = jax.lax.dot_general(x, up_w, dn, preferred_element_type=jnp.float32)
    h = (g * jax.lax.logistic(g)) * u
    o = jax.lax.dot_general(h, down_w, dn, preferred_element_type=jnp.float32)
    contrib = o * w_e

    first = jnp.logical_and(e == 0, f == 0)

    @pl.when(first)
    def _init():
        out_ref[pl.ds(t * BT, BT), :] = contrib

    @pl.when(jnp.logical_not(first))
    def _acc():
        out_ref[pl.ds(t * BT, BT), :] += contrib


def kernel(x, router_logits, gate_up_proj, down_proj, top_k=2):
    gate_p = gate_up_proj[:, :F, :]
    up_p = gate_up_proj[:, F:, :]
    grid = (E, F // BF, T // BT)
    out = pl.pallas_call(
        _moe_dense_kernel,
        grid=grid,
        in_specs=[
            pl.BlockSpec((BT, E), lambda e, f, t: (t, 0)),
            pl.BlockSpec((BT, H), lambda e, f, t: (t, 0)),
            pl.BlockSpec((1, BF, H), lambda e, f, t: (e, f, 0)),
            pl.BlockSpec((1, BF, H), lambda e, f, t: (e, f, 0)),
            pl.BlockSpec((1, H, BF), lambda e, f, t: (e, 0, f)),
        ],
        out_specs=pl.BlockSpec((T, H), lambda e, f, t: (0, 0)),
        out_shape=jax.ShapeDtypeStruct((T, H), jnp.float32),
        compiler_params=pltpu.CompilerParams(
            dimension_semantics=("arbitrary", "arbitrary", "arbitrary"),
        ),
    )(router_logits, x, gate_p, up_p, down_proj)
    scale = jnp.asarray(top_k, jnp.float32) / jnp.float32(2)
    return out * scale
